# baseline (speedup 1.0000x reference)
import functools

import jax
import jax.numpy as jnp
from jax import lax
from jax.experimental import pallas as pl
from jax.experimental.pallas import tpu as pltpu
from jax.experimental.pallas import tpu_sc as plsc

HASH_SIZE = 2 ** 22
CELL_SIZE = 0.001
DIM = 4

_M1 = 2246822507
_M2 = 3266489909
_M1L, _M1H = _M1 & 0xFFFF, _M1 >> 16
_M2L, _M2H = _M2 & 0xFFFF, _M2 >> 16

_NC = 2
_NS = 16
_NW = _NC * _NS
_L = 16

_CHUNK = 1024
_STREAM = 1024


def _i32(v):
    return jnp.int32(v)


def _hash16(ix, iy):
    mask16 = _i32(0xFFFF)
    c13 = _i32(13)
    c16 = _i32(16)
    c3 = _i32(3)
    a = ix * _i32(_M1L)
    b = ix * _i32(_M1H)
    l0 = a & mask16
    s1 = (a >> c16) + (b & mask16)
    l1 = s1 & mask16
    l2 = (b >> c16) + (s1 >> c16)
    q0 = l0 ^ (((l0 >> c13) | (l1 << c3)) & mask16)
    q1 = l1 ^ (((l1 >> c13) | (l2 << c3)) & mask16)
    c = iy * _i32(_M2L)
    d = iy * _i32(_M2H)
    g0 = c & mask16
    t1 = (c >> c16) + (d & mask16)
    g1 = t1 & mask16
    g2 = (d >> c16) + (t1 >> c16)
    u0 = q0 + g0
    r0 = u0 & mask16
    u1 = q1 + g1 + (u0 >> c16)
    r1 = u1 & mask16
    r2 = l2 + g2 + (u1 >> c16)
    s0 = r0 ^ r1
    s1b = r1 ^ r2
    return s0 | ((s1b & _i32(0x3F)) << c16)


def _make_kernel(n):
    per_w = n // _NW
    n_chunks = per_w // _CHUNK
    n_streams = _CHUNK * DIM // _STREAM
    mesh = plsc.VectorSubcoreMesh(
        core_axis_name="c", subcore_axis_name="s",
        num_cores=_NC, num_subcores=_NS)

    @functools.partial(
        pl.kernel,
        mesh=mesh,
        out_type=jax.ShapeDtypeStruct((n * DIM,), jnp.float32),
        scratch_types=[
            pltpu.VMEM((2, _CHUNK * 2), jnp.float32),
            pltpu.VMEM((2, n_streams, _STREAM), jnp.int32),
            pltpu.VMEM((2, _CHUNK * DIM), jnp.float32),
            pltpu.SemaphoreType.DMA,
            pltpu.SemaphoreType.DMA,
            pltpu.SemaphoreType.DMA,
            pltpu.SemaphoreType.DMA,
        ],
        compiler_params=pltpu.CompilerParams(
            needs_layout_passes=False, use_tc_tiling_on_sc=False),
    )
    def k(pos_hbm, grid_hbm, out_hbm, pos_v, idx_v, stage_v,
          gsem_a, gsem_b, osem, psem):
        wid = lax.axis_index("s") * _i32(_NC) + lax.axis_index("c")
        base = wid * _i32(per_w)

        def start_pos(ci, b):
            off = base + ci * _i32(_CHUNK)
            pltpu.async_copy(
                pos_hbm.at[pl.ds(off * _i32(2), _CHUNK * 2)],
                pos_v.at[b], psem)

        def wait_pos(b):
            pltpu.make_async_copy(
                pos_hbm.at[pl.ds(base * _i32(2), _CHUNK * 2)],
                pos_v.at[b], psem).wait()

        def compute_chunk(ci, b):
            def vec_body(kk, carry2):
                blk = kk >> _i32(3)
                r0 = (kk & _i32(7)) * _i32(_L)
                po = blk * _i32(256) + r0
                x = pos_v[b, pl.ds(po, _L)]
                y = pos_v[b, pl.ds(po + _i32(128), _L)]
                ix = (x / CELL_SIZE).astype(jnp.int32)
                iy = (y / CELL_SIZE).astype(jnp.int32)
                idx = _hash16(ix, iy)
                w0 = ((idx >> _i32(7)) << _i32(9)) + (idx & _i32(127))
                wbase = blk * _i32(512) + r0
                lgs = _STREAM.bit_length() - 1
                for dcomp in range(DIM):
                    w = wbase + _i32(128 * dcomp)
                    idx_v[b, w >> _i32(lgs), pl.ds(w & _i32(_STREAM - 1), _L)] = (
                        w0 + _i32(128 * dcomp))
                return carry2

            lax.fori_loop(0, _CHUNK // _L, vec_body, _i32(0), unroll=4)

        def fire_streams(b, gsem):
            for j in range(n_streams):
                pltpu.async_copy(
                    grid_hbm.at[idx_v.at[b, _i32(j)]],
                    stage_v.at[b, pl.ds(j * _STREAM, _STREAM)],
                    gsem,
                )

        def drain_streams(b, gsem):
            for j in range(n_streams):
                pltpu.make_async_copy(
                    grid_hbm.at[pl.ds(_i32(0), _STREAM)],
                    stage_v.at[b, pl.ds(j * _STREAM, _STREAM)],
                    gsem,
                ).wait()

        def start_out(ci, b):
            off = base + ci * _i32(_CHUNK)
            pltpu.async_copy(
                stage_v.at[b],
                out_hbm.at[pl.ds(off * _i32(DIM), _CHUNK * DIM)],
                osem,
            )

        def wait_out(b):
            pltpu.make_async_copy(
                stage_v.at[b],
                out_hbm.at[pl.ds(base * _i32(DIM), _CHUNK * DIM)],
                osem,
            ).wait()

        def half_body(ci, b, gsem_mine, gsem_other):
            nb = 1 - b
            wait_pos(b)

            @pl.when(ci < _i32(n_chunks - 1))
            def _():
                start_pos(ci + _i32(1), nb)

            compute_chunk(ci, b)

            @pl.when(ci > _i32(1))
            def _():
                wait_out(b)

            fire_streams(b, gsem_mine)

            @pl.when(ci > _i32(0))
            def _():
                drain_streams(nb, gsem_other)
                start_out(ci - _i32(1), nb)

        def pair_body(i, carry):
            ci0 = i * _i32(2)
            half_body(ci0, 0, gsem_a, gsem_b)
            half_body(ci0 + _i32(1), 1, gsem_b, gsem_a)
            return carry

        start_pos(_i32(0), _i32(0))
        lax.fori_loop(0, n_chunks // 2, pair_body, _i32(0))
        drain_streams(1, gsem_b)
        wait_out(0)
        start_out(_i32(n_chunks - 1), 1)
        wait_out(1)

    return k


def kernel(positions, grid):
    n = positions.shape[0]
    with jax.enable_x64(False):
        posv = positions.reshape(
            n // 128, 128, 2).transpose(0, 2, 1).reshape(-1)
        gridv = grid.reshape(HASH_SIZE // 128, 128, DIM)
        gridv = gridv.transpose(0, 2, 1).reshape(-1)
        out1d = _make_kernel(n)(posv, gridv)
        out = out1d.reshape(
            n // 128, DIM, 128).transpose(0, 2, 1).reshape(n, DIM)
    return out


# speedup vs baseline: 1.0053x; 1.0053x over previous
import functools

import jax
import jax.numpy as jnp
from jax import lax
from jax.experimental import pallas as pl
from jax.experimental.pallas import tpu as pltpu
from jax.experimental.pallas import tpu_sc as plsc

HASH_SIZE = 2 ** 22
CELL_SIZE = 0.001
DIM = 4

_M1 = 2246822507
_M2 = 3266489909
_M1L, _M1H = _M1 & 0xFFFF, _M1 >> 16
_M2L, _M2H = _M2 & 0xFFFF, _M2 >> 16

_NC = 2
_NS = 16
_NW = _NC * _NS
_L = 16

_CHUNK = 2048
_STREAM = 1024


def _i32(v):
    return jnp.int32(v)


def _hash16(ix, iy):
    mask16 = _i32(0xFFFF)
    c13 = _i32(13)
    c16 = _i32(16)
    c3 = _i32(3)
    a = ix * _i32(_M1L)
    b = ix * _i32(_M1H)
    l0 = a & mask16
    s1 = (a >> c16) + (b & mask16)
    l1 = s1 & mask16
    l2 = (b >> c16) + (s1 >> c16)
    q0 = l0 ^ (((l0 >> c13) | (l1 << c3)) & mask16)
    q1 = l1 ^ (((l1 >> c13) | (l2 << c3)) & mask16)
    c = iy * _i32(_M2L)
    d = iy * _i32(_M2H)
    g0 = c & mask16
    t1 = (c >> c16) + (d & mask16)
    g1 = t1 & mask16
    g2 = (d >> c16) + (t1 >> c16)
    u0 = q0 + g0
    r0 = u0 & mask16
    u1 = q1 + g1 + (u0 >> c16)
    r1 = u1 & mask16
    r2 = l2 + g2 + (u1 >> c16)
    s0 = r0 ^ r1
    s1b = r1 ^ r2
    return s0 | ((s1b & _i32(0x3F)) << c16)


def _make_kernel(n):
    per_w = n // _NW
    n_chunks = per_w // _CHUNK
    n_streams = _CHUNK * DIM // _STREAM
    mesh = plsc.VectorSubcoreMesh(
        core_axis_name="c", subcore_axis_name="s",
        num_cores=_NC, num_subcores=_NS)

    @functools.partial(
        pl.kernel,
        mesh=mesh,
        out_type=jax.ShapeDtypeStruct((n * DIM,), jnp.float32),
        scratch_types=[
            pltpu.VMEM((2, _CHUNK * 2), jnp.float32),
            pltpu.VMEM((2, n_streams, _STREAM), jnp.int32),
            pltpu.VMEM((2, _CHUNK * DIM), jnp.float32),
            pltpu.SemaphoreType.DMA,
            pltpu.SemaphoreType.DMA,
            pltpu.SemaphoreType.DMA,
            pltpu.SemaphoreType.DMA,
        ],
        compiler_params=pltpu.CompilerParams(
            needs_layout_passes=False, use_tc_tiling_on_sc=False),
    )
    def k(pos_hbm, grid_hbm, out_hbm, pos_v, idx_v, stage_v,
          gsem_a, gsem_b, osem, psem):
        wid = lax.axis_index("s") * _i32(_NC) + lax.axis_index("c")
        base = wid * _i32(per_w)

        def start_pos(ci, b):
            off = base + ci * _i32(_CHUNK)
            pltpu.async_copy(
                pos_hbm.at[pl.ds(off * _i32(2), _CHUNK * 2)],
                pos_v.at[b], psem)

        def wait_pos(b):
            pltpu.make_async_copy(
                pos_hbm.at[pl.ds(base * _i32(2), _CHUNK * 2)],
                pos_v.at[b], psem).wait()

        def compute_chunk(ci, b):
            def vec_body(kk, carry2):
                blk = kk >> _i32(3)
                r0 = (kk & _i32(7)) * _i32(_L)
                po = blk * _i32(256) + r0
                x = pos_v[b, pl.ds(po, _L)]
                y = pos_v[b, pl.ds(po + _i32(128), _L)]
                ix = (x / CELL_SIZE).astype(jnp.int32)
                iy = (y / CELL_SIZE).astype(jnp.int32)
                idx = _hash16(ix, iy)
                w0 = ((idx >> _i32(7)) << _i32(9)) + (idx & _i32(127))
                wbase = blk * _i32(512) + r0
                lgs = _STREAM.bit_length() - 1
                for dcomp in range(DIM):
                    w = wbase + _i32(128 * dcomp)
                    idx_v[b, w >> _i32(lgs), pl.ds(w & _i32(_STREAM - 1), _L)] = (
                        w0 + _i32(128 * dcomp))
                return carry2

            lax.fori_loop(0, _CHUNK // _L, vec_body, _i32(0), unroll=4)

        def fire_streams(b, gsem):
            for j in range(n_streams):
                pltpu.async_copy(
                    grid_hbm.at[idx_v.at[b, _i32(j)]],
                    stage_v.at[b, pl.ds(j * _STREAM, _STREAM)],
                    gsem,
                )

        def drain_streams(b, gsem):
            for j in range(n_streams):
                pltpu.make_async_copy(
                    grid_hbm.at[pl.ds(_i32(0), _STREAM)],
                    stage_v.at[b, pl.ds(j * _STREAM, _STREAM)],
                    gsem,
                ).wait()

        def start_out(ci, b):
            off = base + ci * _i32(_CHUNK)
            pltpu.async_copy(
                stage_v.at[b],
                out_hbm.at[pl.ds(off * _i32(DIM), _CHUNK * DIM)],
                osem,
            )

        def wait_out(b):
            pltpu.make_async_copy(
                stage_v.at[b],
                out_hbm.at[pl.ds(base * _i32(DIM), _CHUNK * DIM)],
                osem,
            ).wait()

        def half_body(ci, b, gsem_mine, gsem_other):
            nb = 1 - b
            wait_pos(b)

            @pl.when(ci < _i32(n_chunks - 1))
            def _():
                start_pos(ci + _i32(1), nb)

            compute_chunk(ci, b)

            @pl.when(ci > _i32(1))
            def _():
                wait_out(b)

            fire_streams(b, gsem_mine)

            @pl.when(ci > _i32(0))
            def _():
                drain_streams(nb, gsem_other)
                start_out(ci - _i32(1), nb)

        def pair_body(i, carry):
            ci0 = i * _i32(2)
            half_body(ci0, 0, gsem_a, gsem_b)
            half_body(ci0 + _i32(1), 1, gsem_b, gsem_a)
            return carry

        start_pos(_i32(0), _i32(0))
        lax.fori_loop(0, n_chunks // 2, pair_body, _i32(0))
        drain_streams(1, gsem_b)
        wait_out(0)
        start_out(_i32(n_chunks - 1), 1)
        wait_out(1)

    return k


def kernel(positions, grid):
    n = positions.shape[0]
    with jax.enable_x64(False):
        posv = positions.reshape(
            n // 128, 128, 2).transpose(0, 2, 1).reshape(-1)
        gridv = grid.reshape(HASH_SIZE // 128, 128, DIM)
        gridv = gridv.transpose(0, 2, 1).reshape(-1)
        out1d = _make_kernel(n)(posv, gridv)
        out = out1d.reshape(
            n // 128, DIM, 128).transpose(0, 2, 1).reshape(n, DIM)
    return out
